# baseline (speedup 1.0000x reference)
import jax
import jax.numpy as jnp
from jax import lax
from jax.experimental import pallas as pl
from jax.experimental.pallas import tpu as pltpu
from jax.experimental.pallas import tpu_sc as plsc

N = 10000
E = 320000
D_IN = 128
D_H = 128
D_OUT = 64

NC = 2
NS = 16
NW = NC * NS

CHUNK = 128
N_PAD = 10240
ROWS_PER_TILE = N_PAD // NS
CHUNKS_PER_TILE = 80
EDGES_PER_TILE = CHUNKS_PER_TILE * CHUNK
E_PAD = NW * EDGES_PER_TILE
NBUF = 2
WCHUNKS = 16
WINDOWS = CHUNKS_PER_TILE // WCHUNKS

_MESH = plsc.VectorSubcoreMesh(
    core_axis_name="c", subcore_axis_name="s", num_cores=NC, num_subcores=NS)
_PARAMS = pltpu.CompilerParams(use_tc_tiling_on_sc=False)


def _deg_body(src_ref, dst_ref, ones_ref, zeros_ref, do_out, di_out,
              sidx, didx, ones_v, sem, do_acc, di_acc):
    c = lax.axis_index("c")
    s = lax.axis_index("s")
    wid = c * NS + s
    row0 = s * ROWS_PER_TILE
    pltpu.sync_copy(zeros_ref.at[pl.ds(row0, ROWS_PER_TILE)],
                    do_acc.at[pl.ds(row0, ROWS_PER_TILE)])
    pltpu.sync_copy(zeros_ref.at[pl.ds(row0, ROWS_PER_TILE)],
                    di_acc.at[pl.ds(row0, ROWS_PER_TILE)])
    pltpu.sync_copy(ones_ref, ones_v)
    pltpu.sync_copy(src_ref.at[pl.ds(wid * CHUNKS_PER_TILE, CHUNKS_PER_TILE)], sidx)
    pltpu.sync_copy(dst_ref.at[pl.ds(wid * CHUNKS_PER_TILE, CHUNKS_PER_TILE)], didx)
    plsc.subcore_barrier()

    def issue(i, carry):
        pltpu.async_copy(ones_v, do_acc.at[sidx.at[i]], sem.at[0], add=True)
        pltpu.async_copy(ones_v, di_acc.at[didx.at[i]], sem.at[1], add=True)
        return carry

    lax.fori_loop(0, CHUNKS_PER_TILE, issue, 0)

    def drain(i, carry):
        pltpu.make_async_copy(ones_v, do_acc.at[sidx.at[i]], sem.at[0]).wait()
        pltpu.make_async_copy(ones_v, di_acc.at[didx.at[i]], sem.at[1]).wait()
        return carry

    lax.fori_loop(0, CHUNKS_PER_TILE, drain, 0)
    plsc.subcore_barrier()
    pltpu.sync_copy(do_acc.at[pl.ds(row0, ROWS_PER_TILE)],
                    do_out.at[c, pl.ds(row0, ROWS_PER_TILE)])
    pltpu.sync_copy(di_acc.at[pl.ds(row0, ROWS_PER_TILE)],
                    di_out.at[c, pl.ds(row0, ROWS_PER_TILE)])


def _make_deg_kernel():
    return pl.kernel(
        _deg_body,
        out_type=(jax.ShapeDtypeStruct((NC, N_PAD, 16), jnp.float32),
                  jax.ShapeDtypeStruct((NC, N_PAD, 16), jnp.float32)),
        mesh=_MESH,
        compiler_params=_PARAMS,
        scratch_types=[
            pltpu.VMEM((CHUNKS_PER_TILE, CHUNK), jnp.int32),
            pltpu.VMEM((CHUNKS_PER_TILE, CHUNK), jnp.int32),
            pltpu.VMEM((CHUNK, 16), jnp.float32),
            pltpu.SemaphoreType.DMA((2,)),
            pltpu.VMEM_SHARED((N_PAD, 16), jnp.float32),
            pltpu.VMEM_SHARED((N_PAD, 16), jnp.float32),
        ],
    )


def _prop_body(table_ref, src_ref, dst_ref, zeros_ref, out_ref,
               sidx, didx, rows, gsem, acc):
    c = lax.axis_index("c")
    s = lax.axis_index("s")
    wid = c * NS + s
    row0 = s * ROWS_PER_TILE
    pltpu.sync_copy(zeros_ref.at[pl.ds(row0, ROWS_PER_TILE)],
                    acc.at[pl.ds(row0, ROWS_PER_TILE)])
    plsc.subcore_barrier()

    def gather(i, b):
        pltpu.async_copy(table_ref.at[sidx.at[i]], rows.at[b], gsem.at[b])

    def window(w, carry):
        base = wid * CHUNKS_PER_TILE + w * WCHUNKS
        pltpu.sync_copy(src_ref.at[pl.ds(base, WCHUNKS)], sidx)
        pltpu.sync_copy(dst_ref.at[pl.ds(base, WCHUNKS)], didx)
        for b in range(NBUF):
            gather(b, b)

        def group(g, carry2):
            for b in range(NBUF):
                i = g * NBUF + b
                pltpu.make_async_copy(table_ref.at[sidx.at[i]], rows.at[b],
                                      gsem.at[b]).wait()
                pltpu.sync_copy(rows.at[b], acc.at[didx.at[i]], add=True)
                j = i + NBUF

                @pl.when(j < WCHUNKS)
                def _():
                    gather(j, b)
            return carry2

        lax.fori_loop(0, WCHUNKS // NBUF, group, 0)
        return carry

    lax.fori_loop(0, WINDOWS, window, 0)
    plsc.subcore_barrier()
    pltpu.sync_copy(acc.at[pl.ds(row0, ROWS_PER_TILE)],
                    out_ref.at[c, pl.ds(row0, ROWS_PER_TILE)])


def _make_prop_kernel(d):
    return pl.kernel(
        _prop_body,
        out_type=jax.ShapeDtypeStruct((NC, N_PAD, d), jnp.float32),
        mesh=_MESH,
        compiler_params=_PARAMS,
        scratch_types=[
            pltpu.VMEM((WCHUNKS, CHUNK), jnp.int32),
            pltpu.VMEM((WCHUNKS, CHUNK), jnp.int32),
            pltpu.VMEM((NBUF, CHUNK, d), jnp.float32),
            pltpu.SemaphoreType.DMA((NBUF,)),
            pltpu.VMEM_SHARED((N_PAD, d), jnp.float32),
        ],
    )


def _norm_from(parts_ref):
    deg = parts_ref[0, :, 0:1] + parts_ref[1, :, 0:1]
    return jnp.where(deg > 0.0, lax.rsqrt(jnp.maximum(deg, 1.0)), 0.0)


def _tc1_body(do_ref, feat_ref, w1_ref, y1_ref):
    nsrc = _norm_from(do_ref)
    y1_ref[...] = jnp.dot(feat_ref[...] * nsrc, w1_ref[...],
                          preferred_element_type=jnp.float32)


def _tc2_body(p_ref, do_ref, di_ref, b1_ref, w2_ref, y2_ref):
    nsrc = _norm_from(do_ref)
    ndst = _norm_from(di_ref)
    agg = (p_ref[0] + p_ref[1]) * ndst + b1_ref[...]
    h = jnp.maximum(agg, 0.0)
    y2_ref[...] = jnp.dot(h * nsrc, w2_ref[...],
                          preferred_element_type=jnp.float32)


def _tc3_body(p_ref, di_ref, b2_ref, o_ref):
    ndst = _norm_from(di_ref)
    o_ref[...] = (p_ref[0] + p_ref[1]) * ndst + b2_ref[...]


_R = 1024
_GRID = N_PAD // _R


def _deg_spec():
    return pl.BlockSpec((NC, _R, 16), lambda i: (0, i, 0))


def kernel(features, edge_index, W1, b1, W2, b2):
    src = edge_index[0]
    dst = edge_index[1]
    pad_idx = jnp.full((E_PAD - E,), N, dtype=jnp.int32)
    src_p = jnp.concatenate([src, pad_idx]).reshape(NW * CHUNKS_PER_TILE, CHUNK)
    dst_p = jnp.concatenate([dst, pad_idx]).reshape(NW * CHUNKS_PER_TILE, CHUNK)
    feat_p = jnp.zeros((N_PAD, D_IN), jnp.float32).at[:N].set(features)
    ones16 = jnp.ones((CHUNK, 16), jnp.float32)
    zeros16 = jnp.zeros((N_PAD, 16), jnp.float32)
    zeros_h = jnp.zeros((N_PAD, D_H), jnp.float32)
    zeros_o = jnp.zeros((N_PAD, D_OUT), jnp.float32)

    do_part, di_part = _make_deg_kernel()(src_p, dst_p, ones16, zeros16)

    y1 = pl.pallas_call(
        _tc1_body,
        grid=(_GRID,),
        in_specs=[
            _deg_spec(),
            pl.BlockSpec((_R, D_IN), lambda i: (i, 0)),
            pl.BlockSpec((D_IN, D_H), lambda i: (0, 0)),
        ],
        out_specs=pl.BlockSpec((_R, D_H), lambda i: (i, 0)),
        out_shape=jax.ShapeDtypeStruct((N_PAD, D_H), jnp.float32),
    )(do_part, feat_p, W1)

    p1 = _make_prop_kernel(D_H)(y1, src_p, dst_p, zeros_h)

    y2 = pl.pallas_call(
        _tc2_body,
        grid=(_GRID,),
        in_specs=[
            pl.BlockSpec((NC, _R, D_H), lambda i: (0, i, 0)),
            _deg_spec(),
            _deg_spec(),
            pl.BlockSpec((1, D_H), lambda i: (0, 0)),
            pl.BlockSpec((D_H, D_OUT), lambda i: (0, 0)),
        ],
        out_specs=pl.BlockSpec((_R, D_OUT), lambda i: (i, 0)),
        out_shape=jax.ShapeDtypeStruct((N_PAD, D_OUT), jnp.float32),
    )(p1, do_part, di_part, b1.reshape(1, D_H), W2)

    p2 = _make_prop_kernel(D_OUT)(y2, src_p, dst_p, zeros_o)

    out = pl.pallas_call(
        _tc3_body,
        grid=(_GRID,),
        in_specs=[
            pl.BlockSpec((NC, _R, D_OUT), lambda i: (0, i, 0)),
            _deg_spec(),
            pl.BlockSpec((1, D_OUT), lambda i: (0, 0)),
        ],
        out_specs=pl.BlockSpec((_R, D_OUT), lambda i: (i, 0)),
        out_shape=jax.ShapeDtypeStruct((N_PAD, D_OUT), jnp.float32),
    )(p2, di_part, b2.reshape(1, D_OUT))

    return out[:N]


# speedup vs baseline: 1.9931x; 1.9931x over previous
import jax
import jax.numpy as jnp
from jax import lax
from jax.experimental import pallas as pl
from jax.experimental.pallas import tpu as pltpu
from jax.experimental.pallas import tpu_sc as plsc

N = 10000
E = 320000
D_IN = 128
D_H = 128
D_OUT = 64

NC = 2
NS = 16
NW = NC * NS

CHUNK = 128
N_PAD = 10240
ROWS_PER_TILE = N_PAD // NS
CHUNKS_PER_TILE = 80
EDGES_PER_TILE = CHUNKS_PER_TILE * CHUNK
E_PAD = NW * EDGES_PER_TILE
NBUF = 4
WCHUNKS = 16
WINDOWS = CHUNKS_PER_TILE // WCHUNKS

_MESH = plsc.VectorSubcoreMesh(
    core_axis_name="c", subcore_axis_name="s", num_cores=NC, num_subcores=NS)
_PARAMS = pltpu.CompilerParams(use_tc_tiling_on_sc=False)


def _deg_body(src_ref, dst_ref, ones_ref, zeros_ref, do_out, di_out,
              sidx, didx, ones_v, sem, do_acc, di_acc):
    c = lax.axis_index("c")
    s = lax.axis_index("s")
    wid = c * NS + s
    row0 = s * ROWS_PER_TILE
    pltpu.sync_copy(zeros_ref.at[pl.ds(row0, ROWS_PER_TILE)],
                    do_acc.at[pl.ds(row0, ROWS_PER_TILE)])
    pltpu.sync_copy(zeros_ref.at[pl.ds(row0, ROWS_PER_TILE)],
                    di_acc.at[pl.ds(row0, ROWS_PER_TILE)])
    pltpu.sync_copy(ones_ref, ones_v)
    pltpu.sync_copy(src_ref.at[pl.ds(wid * CHUNKS_PER_TILE, CHUNKS_PER_TILE)], sidx)
    pltpu.sync_copy(dst_ref.at[pl.ds(wid * CHUNKS_PER_TILE, CHUNKS_PER_TILE)], didx)
    plsc.subcore_barrier()

    def issue(i, carry):
        pltpu.async_copy(ones_v, do_acc.at[sidx.at[i]], sem.at[0], add=True)
        pltpu.async_copy(ones_v, di_acc.at[didx.at[i]], sem.at[1], add=True)
        return carry

    lax.fori_loop(0, CHUNKS_PER_TILE, issue, 0)

    def drain(i, carry):
        pltpu.make_async_copy(ones_v, do_acc.at[sidx.at[i]], sem.at[0]).wait()
        pltpu.make_async_copy(ones_v, di_acc.at[didx.at[i]], sem.at[1]).wait()
        return carry

    lax.fori_loop(0, CHUNKS_PER_TILE, drain, 0)
    plsc.subcore_barrier()
    pltpu.sync_copy(do_acc.at[pl.ds(row0, ROWS_PER_TILE)],
                    do_out.at[c, pl.ds(row0, ROWS_PER_TILE)])
    pltpu.sync_copy(di_acc.at[pl.ds(row0, ROWS_PER_TILE)],
                    di_out.at[c, pl.ds(row0, ROWS_PER_TILE)])


def _make_deg_kernel():
    return pl.kernel(
        _deg_body,
        out_type=(jax.ShapeDtypeStruct((NC, N_PAD, 16), jnp.float32),
                  jax.ShapeDtypeStruct((NC, N_PAD, 16), jnp.float32)),
        mesh=_MESH,
        compiler_params=_PARAMS,
        scratch_types=[
            pltpu.VMEM((CHUNKS_PER_TILE, CHUNK), jnp.int32),
            pltpu.VMEM((CHUNKS_PER_TILE, CHUNK), jnp.int32),
            pltpu.VMEM((CHUNK, 16), jnp.float32),
            pltpu.SemaphoreType.DMA((2,)),
            pltpu.VMEM_SHARED((N_PAD, 16), jnp.float32),
            pltpu.VMEM_SHARED((N_PAD, 16), jnp.float32),
        ],
    )


def _run_edge_windows(src_ref, dst_ref, table_sp, acc, sidx, didx, rows, gsem,
                      chunk_base, n_windows):
    def gather(i, b):
        pltpu.async_copy(table_sp.at[sidx.at[i]], rows.at[b], gsem.at[b])

    def window(w, carry):
        base = chunk_base + w * WCHUNKS
        pltpu.sync_copy(src_ref.at[pl.ds(base, WCHUNKS)], sidx)
        pltpu.sync_copy(dst_ref.at[pl.ds(base, WCHUNKS)], didx)
        for b in range(NBUF):
            gather(b, b)

        def group(g, carry2):
            for b in range(NBUF):
                i = g * NBUF + b
                pltpu.make_async_copy(table_sp.at[sidx.at[i]], rows.at[b],
                                      gsem.at[b]).wait()
                pltpu.sync_copy(rows.at[b], acc.at[didx.at[i]], add=True)
                j = i + NBUF

                @pl.when(j < WCHUNKS)
                def _():
                    gather(j, b)
            return carry2

        lax.fori_loop(0, WCHUNKS // NBUF, group, 0)
        return carry

    lax.fori_loop(0, n_windows, window, 0)


def _prop_split_body(table_ref, src_ref, dst_ref, zeros_ref, out_ref,
                     sidx, didx, rows, gsem, table_sp, acc):
    c = lax.axis_index("c")
    s = lax.axis_index("s")
    row0 = s * ROWS_PER_TILE
    pltpu.sync_copy(table_ref.at[c, pl.ds(row0, ROWS_PER_TILE)],
                    table_sp.at[pl.ds(row0, ROWS_PER_TILE)])
    pltpu.sync_copy(zeros_ref.at[pl.ds(row0, ROWS_PER_TILE)],
                    acc.at[pl.ds(row0, ROWS_PER_TILE)])
    plsc.subcore_barrier()
    _run_edge_windows(src_ref, dst_ref, table_sp, acc, sidx, didx, rows, gsem,
                      s * (2 * CHUNKS_PER_TILE), 2 * WINDOWS)
    plsc.subcore_barrier()
    pltpu.sync_copy(acc.at[pl.ds(row0, ROWS_PER_TILE)],
                    out_ref.at[c, pl.ds(row0, ROWS_PER_TILE)])


def _prop_full_body(table_ref, src_ref, dst_ref, zeros_ref, out_ref,
                    sidx, didx, rows, gsem, table_sp, acc):
    c = lax.axis_index("c")
    s = lax.axis_index("s")
    wid = c * NS + s
    row0 = s * ROWS_PER_TILE
    pltpu.sync_copy(table_ref.at[pl.ds(row0, ROWS_PER_TILE)],
                    table_sp.at[pl.ds(row0, ROWS_PER_TILE)])
    pltpu.sync_copy(zeros_ref.at[pl.ds(row0, ROWS_PER_TILE)],
                    acc.at[pl.ds(row0, ROWS_PER_TILE)])
    plsc.subcore_barrier()
    _run_edge_windows(src_ref, dst_ref, table_sp, acc, sidx, didx, rows, gsem,
                      wid * CHUNKS_PER_TILE, WINDOWS)
    plsc.subcore_barrier()
    pltpu.sync_copy(acc.at[pl.ds(row0, ROWS_PER_TILE)],
                    out_ref.at[c, pl.ds(row0, ROWS_PER_TILE)])


def _make_prop_kernel(body, d):
    return pl.kernel(
        body,
        out_type=jax.ShapeDtypeStruct((NC, N_PAD, d), jnp.float32),
        mesh=_MESH,
        compiler_params=_PARAMS,
        scratch_types=[
            pltpu.VMEM((WCHUNKS, CHUNK), jnp.int32),
            pltpu.VMEM((WCHUNKS, CHUNK), jnp.int32),
            pltpu.VMEM((NBUF, CHUNK, d), jnp.float32),
            pltpu.SemaphoreType.DMA((NBUF,)),
            pltpu.VMEM_SHARED((N_PAD, d), jnp.float32),
            pltpu.VMEM_SHARED((N_PAD, d), jnp.float32),
        ],
    )


def _norm_from(parts_ref):
    deg = parts_ref[0, :, 0:1] + parts_ref[1, :, 0:1]
    return jnp.where(deg > 0.0, lax.rsqrt(jnp.maximum(deg, 1.0)), 0.0)


def _tc1_body(do_ref, feat_ref, w1_ref, y1_ref):
    nsrc = _norm_from(do_ref)
    y = jnp.dot(feat_ref[...] * nsrc, w1_ref[...],
                preferred_element_type=jnp.float32)
    y1_ref[0] = y[:, :D_H // 2]
    y1_ref[1] = y[:, D_H // 2:]


def _tc2_body(p_ref, do_ref, di_ref, b1_ref, w2_ref, y2_ref):
    nsrc = _norm_from(do_ref)
    ndst = _norm_from(di_ref)
    agg = jnp.concatenate([p_ref[0], p_ref[1]], axis=-1) * ndst + b1_ref[...]
    h = jnp.maximum(agg, 0.0)
    y2_ref[...] = jnp.dot(h * nsrc, w2_ref[...],
                          preferred_element_type=jnp.float32)


def _tc3_body(p_ref, di_ref, b2_ref, o_ref):
    ndst = _norm_from(di_ref)
    o_ref[...] = (p_ref[0] + p_ref[1]) * ndst + b2_ref[...]


_R = 1024
_GRID = N_PAD // _R


def _deg_spec():
    return pl.BlockSpec((NC, _R, 16), lambda i: (0, i, 0))


def kernel(features, edge_index, W1, b1, W2, b2):
    src = edge_index[0]
    dst = edge_index[1]
    pad_idx = jnp.full((E_PAD - E,), N, dtype=jnp.int32)
    src_p = jnp.concatenate([src, pad_idx]).reshape(NW * CHUNKS_PER_TILE, CHUNK)
    dst_p = jnp.concatenate([dst, pad_idx]).reshape(NW * CHUNKS_PER_TILE, CHUNK)
    feat_p = jnp.zeros((N_PAD, D_IN), jnp.float32).at[:N].set(features)
    ones16 = jnp.ones((CHUNK, 16), jnp.float32)
    zeros16 = jnp.zeros((N_PAD, 16), jnp.float32)
    zeros_o = jnp.zeros((N_PAD, D_OUT), jnp.float32)

    do_part, di_part = _make_deg_kernel()(src_p, dst_p, ones16, zeros16)

    y1 = pl.pallas_call(
        _tc1_body,
        grid=(_GRID,),
        in_specs=[
            _deg_spec(),
            pl.BlockSpec((_R, D_IN), lambda i: (i, 0)),
            pl.BlockSpec((D_IN, D_H), lambda i: (0, 0)),
        ],
        out_specs=pl.BlockSpec((NC, _R, D_H // 2), lambda i: (0, i, 0)),
        out_shape=jax.ShapeDtypeStruct((NC, N_PAD, D_H // 2), jnp.float32),
    )(do_part, feat_p, W1)

    p1 = _make_prop_kernel(_prop_split_body, D_H // 2)(y1, src_p, dst_p, zeros_o)

    y2 = pl.pallas_call(
        _tc2_body,
        grid=(_GRID,),
        in_specs=[
            pl.BlockSpec((NC, _R, D_H // 2), lambda i: (0, i, 0)),
            _deg_spec(),
            _deg_spec(),
            pl.BlockSpec((1, D_H), lambda i: (0, 0)),
            pl.BlockSpec((D_H, D_OUT), lambda i: (0, 0)),
        ],
        out_specs=pl.BlockSpec((_R, D_OUT), lambda i: (i, 0)),
        out_shape=jax.ShapeDtypeStruct((N_PAD, D_OUT), jnp.float32),
    )(p1, do_part, di_part, b1.reshape(1, D_H), W2)

    p2 = _make_prop_kernel(_prop_full_body, D_OUT)(y2, src_p, dst_p, zeros_o)

    out = pl.pallas_call(
        _tc3_body,
        grid=(_GRID,),
        in_specs=[
            pl.BlockSpec((NC, _R, D_OUT), lambda i: (0, i, 0)),
            _deg_spec(),
            pl.BlockSpec((1, D_OUT), lambda i: (0, 0)),
        ],
        out_specs=pl.BlockSpec((_R, D_OUT), lambda i: (i, 0)),
        out_shape=jax.ShapeDtypeStruct((N_PAD, D_OUT), jnp.float32),
    )(p2, di_part, b2.reshape(1, D_OUT))

    return out[:N]


# speedup vs baseline: 2.4726x; 1.2406x over previous
import jax
import jax.numpy as jnp
from jax import lax
from jax.experimental import pallas as pl
from jax.experimental.pallas import tpu as pltpu
from jax.experimental.pallas import tpu_sc as plsc

N = 10000
E = 320000
D_IN = 128
D_H = 128
D_OUT = 64

NC = 2
NS = 16
NW = NC * NS

CHUNK = 128
N_PAD = 10240
ROWS_PER_TILE = N_PAD // NS
CHUNKS_PER_TILE = 80
EDGES_PER_TILE = CHUNKS_PER_TILE * CHUNK
E_PAD = NW * EDGES_PER_TILE
NBUF = 4
WCHUNKS = 16
WINDOWS = CHUNKS_PER_TILE // WCHUNKS

_MESH = plsc.VectorSubcoreMesh(
    core_axis_name="c", subcore_axis_name="s", num_cores=NC, num_subcores=NS)
_PARAMS = pltpu.CompilerParams(use_tc_tiling_on_sc=False)


def _deg_body(src_ref, dst_ref, ones_ref, zeros_ref, do_out, di_out,
              sidx, didx, ones_v, sem, do_acc, di_acc):
    c = lax.axis_index("c")
    s = lax.axis_index("s")
    wid = c * NS + s
    row0 = s * ROWS_PER_TILE
    pltpu.sync_copy(zeros_ref.at[pl.ds(row0, ROWS_PER_TILE)],
                    do_acc.at[pl.ds(row0, ROWS_PER_TILE)])
    pltpu.sync_copy(zeros_ref.at[pl.ds(row0, ROWS_PER_TILE)],
                    di_acc.at[pl.ds(row0, ROWS_PER_TILE)])
    pltpu.sync_copy(ones_ref, ones_v)
    pltpu.sync_copy(src_ref.at[pl.ds(wid * CHUNKS_PER_TILE, CHUNKS_PER_TILE)], sidx)
    pltpu.sync_copy(dst_ref.at[pl.ds(wid * CHUNKS_PER_TILE, CHUNKS_PER_TILE)], didx)
    plsc.subcore_barrier()

    def issue(i, carry):
        pltpu.async_copy(ones_v, do_acc.at[sidx.at[i]], sem.at[0], add=True)
        pltpu.async_copy(ones_v, di_acc.at[didx.at[i]], sem.at[1], add=True)
        return carry

    lax.fori_loop(0, CHUNKS_PER_TILE, issue, 0)

    def drain(i, carry):
        pltpu.make_async_copy(ones_v, do_acc.at[sidx.at[i]], sem.at[0]).wait()
        pltpu.make_async_copy(ones_v, di_acc.at[didx.at[i]], sem.at[1]).wait()
        return carry

    lax.fori_loop(0, CHUNKS_PER_TILE, drain, 0)
    plsc.subcore_barrier()
    pltpu.sync_copy(do_acc.at[pl.ds(row0, ROWS_PER_TILE)],
                    do_out.at[c, pl.ds(row0, ROWS_PER_TILE)])
    pltpu.sync_copy(di_acc.at[pl.ds(row0, ROWS_PER_TILE)],
                    di_out.at[c, pl.ds(row0, ROWS_PER_TILE)])


def _make_deg_kernel():
    return pl.kernel(
        _deg_body,
        out_type=(jax.ShapeDtypeStruct((NC, N_PAD, 16), jnp.float32),
                  jax.ShapeDtypeStruct((NC, N_PAD, 16), jnp.float32)),
        mesh=_MESH,
        compiler_params=_PARAMS,
        scratch_types=[
            pltpu.VMEM((CHUNKS_PER_TILE, CHUNK), jnp.int32),
            pltpu.VMEM((CHUNKS_PER_TILE, CHUNK), jnp.int32),
            pltpu.VMEM((CHUNK, 16), jnp.float32),
            pltpu.SemaphoreType.DMA((2,)),
            pltpu.VMEM_SHARED((N_PAD, 16), jnp.float32),
            pltpu.VMEM_SHARED((N_PAD, 16), jnp.float32),
        ],
    )


def _run_edge_windows(src_ref, dst_ref, table_sp, acc, sidx, didx, rows, gsem,
                      chunk_base, n_windows):
    def gather(i, b):
        pltpu.async_copy(table_sp.at[sidx.at[i]], rows.at[b], gsem.at[b])

    def window(w, carry):
        base = chunk_base + w * WCHUNKS
        pltpu.sync_copy(src_ref.at[pl.ds(base, WCHUNKS)], sidx)
        pltpu.sync_copy(dst_ref.at[pl.ds(base, WCHUNKS)], didx)
        for b in range(NBUF):
            gather(b, b)

        def group(g, carry2):
            for b in range(NBUF):
                i = g * NBUF + b
                pltpu.make_async_copy(table_sp.at[sidx.at[i]], rows.at[b],
                                      gsem.at[b]).wait()
                pltpu.sync_copy(rows.at[b], acc.at[didx.at[i]], add=True)
                j = i + NBUF

                @pl.when(j < WCHUNKS)
                def _():
                    gather(j, b)
            return carry2

        lax.fori_loop(0, WCHUNKS // NBUF, group, 0)
        return carry

    lax.fori_loop(0, n_windows, window, 0)


def _prop_split_body(table_ref, src_ref, dst_ref, zeros_ref, out_ref,
                     sidx, didx, rows, gsem, table_sp, acc):
    c = lax.axis_index("c")
    s = lax.axis_index("s")
    row0 = s * ROWS_PER_TILE
    pltpu.sync_copy(table_ref.at[c, pl.ds(row0, ROWS_PER_TILE)],
                    table_sp.at[pl.ds(row0, ROWS_PER_TILE)])
    pltpu.sync_copy(zeros_ref.at[pl.ds(row0, ROWS_PER_TILE)],
                    acc.at[pl.ds(row0, ROWS_PER_TILE)])
    plsc.subcore_barrier()
    _run_edge_windows(src_ref, dst_ref, table_sp, acc, sidx, didx, rows, gsem,
                      s * (2 * CHUNKS_PER_TILE), 2 * WINDOWS)
    plsc.subcore_barrier()
    pltpu.sync_copy(acc.at[pl.ds(row0, ROWS_PER_TILE)],
                    out_ref.at[c, pl.ds(row0, ROWS_PER_TILE)])


def _prop_full_body(table_ref, src_ref, dst_ref, zeros_ref, out_ref,
                    sidx, didx, rows, gsem, table_sp, acc):
    c = lax.axis_index("c")
    s = lax.axis_index("s")
    wid = c * NS + s
    row0 = s * ROWS_PER_TILE
    pltpu.sync_copy(table_ref.at[pl.ds(row0, ROWS_PER_TILE)],
                    table_sp.at[pl.ds(row0, ROWS_PER_TILE)])
    pltpu.sync_copy(zeros_ref.at[pl.ds(row0, ROWS_PER_TILE)],
                    acc.at[pl.ds(row0, ROWS_PER_TILE)])
    plsc.subcore_barrier()
    _run_edge_windows(src_ref, dst_ref, table_sp, acc, sidx, didx, rows, gsem,
                      wid * CHUNKS_PER_TILE, WINDOWS)
    plsc.subcore_barrier()
    pltpu.sync_copy(acc.at[pl.ds(row0, ROWS_PER_TILE)],
                    out_ref.at[c, pl.ds(row0, ROWS_PER_TILE)])


def _make_prop_kernel(body, d, dtype):
    return pl.kernel(
        body,
        out_type=jax.ShapeDtypeStruct((NC, N_PAD, d), dtype),
        mesh=_MESH,
        compiler_params=_PARAMS,
        scratch_types=[
            pltpu.VMEM((WCHUNKS, CHUNK), jnp.int32),
            pltpu.VMEM((WCHUNKS, CHUNK), jnp.int32),
            pltpu.VMEM((NBUF, CHUNK, d), dtype),
            pltpu.SemaphoreType.DMA((NBUF,)),
            pltpu.VMEM_SHARED((N_PAD, d), dtype),
            pltpu.VMEM_SHARED((N_PAD, d), dtype),
        ],
    )


def _norm_from(parts_ref):
    deg = parts_ref[0, :, 0:1] + parts_ref[1, :, 0:1]
    return jnp.where(deg > 0.0, lax.rsqrt(jnp.maximum(deg, 1.0)), 0.0)


def _tc1_body(do_ref, feat_ref, w1_ref, y1_ref):
    nsrc = _norm_from(do_ref)
    y = jnp.dot(feat_ref[...] * nsrc, w1_ref[...],
                preferred_element_type=jnp.float32).astype(jnp.bfloat16)
    y1_ref[0] = y[:, :D_H // 2]
    y1_ref[1] = y[:, D_H // 2:]


def _tc2_body(p_ref, do_ref, di_ref, b1_ref, w2_ref, y2_ref):
    nsrc = _norm_from(do_ref)
    ndst = _norm_from(di_ref)
    agg = jnp.concatenate([p_ref[0], p_ref[1]],
                          axis=-1).astype(jnp.float32) * ndst + b1_ref[...]
    h = jnp.maximum(agg, 0.0)
    y2_ref[...] = jnp.dot(h * nsrc, w2_ref[...],
                          preferred_element_type=jnp.float32)


def _tc3_body(p_ref, di_ref, b2_ref, o_ref):
    ndst = _norm_from(di_ref)
    o_ref[...] = (p_ref[0] + p_ref[1]) * ndst + b2_ref[...]


_R = 1024
_GRID = N_PAD // _R


def _deg_spec():
    return pl.BlockSpec((NC, _R, 16), lambda i: (0, i, 0))


def kernel(features, edge_index, W1, b1, W2, b2):
    src = edge_index[0]
    dst = edge_index[1]
    pad_idx = jnp.full((E_PAD - E,), N, dtype=jnp.int32)
    src_p = jnp.concatenate([src, pad_idx]).reshape(NW * CHUNKS_PER_TILE, CHUNK)
    dst_p = jnp.concatenate([dst, pad_idx]).reshape(NW * CHUNKS_PER_TILE, CHUNK)
    feat_p = jnp.zeros((N_PAD, D_IN), jnp.float32).at[:N].set(features)
    ones16 = jnp.ones((CHUNK, 16), jnp.float32)
    zeros16 = jnp.zeros((N_PAD, 16), jnp.float32)
    zeros_o = jnp.zeros((N_PAD, D_OUT), jnp.float32)

    do_part, di_part = _make_deg_kernel()(src_p, dst_p, ones16, zeros16)

    y1 = pl.pallas_call(
        _tc1_body,
        grid=(_GRID,),
        in_specs=[
            _deg_spec(),
            pl.BlockSpec((_R, D_IN), lambda i: (i, 0)),
            pl.BlockSpec((D_IN, D_H), lambda i: (0, 0)),
        ],
        out_specs=pl.BlockSpec((NC, _R, D_H // 2), lambda i: (0, i, 0)),
        out_shape=jax.ShapeDtypeStruct((NC, N_PAD, D_H // 2), jnp.bfloat16),
    )(do_part, feat_p, W1)

    zeros_bf = jnp.zeros((N_PAD, D_H // 2), jnp.bfloat16)
    p1 = _make_prop_kernel(_prop_split_body, D_H // 2, jnp.bfloat16)(
        y1, src_p, dst_p, zeros_bf)

    y2 = pl.pallas_call(
        _tc2_body,
        grid=(_GRID,),
        in_specs=[
            pl.BlockSpec((NC, _R, D_H // 2), lambda i: (0, i, 0)),
            _deg_spec(),
            _deg_spec(),
            pl.BlockSpec((1, D_H), lambda i: (0, 0)),
            pl.BlockSpec((D_H, D_OUT), lambda i: (0, 0)),
        ],
        out_specs=pl.BlockSpec((_R, D_OUT), lambda i: (i, 0)),
        out_shape=jax.ShapeDtypeStruct((N_PAD, D_OUT), jnp.float32),
    )(p1, do_part, di_part, b1.reshape(1, D_H), W2)

    p2 = _make_prop_kernel(_prop_full_body, D_OUT, jnp.float32)(
        y2, src_p, dst_p, zeros_o)

    out = pl.pallas_call(
        _tc3_body,
        grid=(_GRID,),
        in_specs=[
            pl.BlockSpec((NC, _R, D_OUT), lambda i: (0, i, 0)),
            _deg_spec(),
            pl.BlockSpec((1, D_OUT), lambda i: (0, 0)),
        ],
        out_specs=pl.BlockSpec((_R, D_OUT), lambda i: (i, 0)),
        out_shape=jax.ShapeDtypeStruct((N_PAD, D_OUT), jnp.float32),
    )(p2, di_part, b2.reshape(1, D_OUT))

    return out[:N]


# speedup vs baseline: 2.8231x; 1.1418x over previous
import jax
import jax.numpy as jnp
from jax import lax
from jax.experimental import pallas as pl
from jax.experimental.pallas import tpu as pltpu
from jax.experimental.pallas import tpu_sc as plsc

N = 10000
E = 320000
D_IN = 128
D_H = 128
D_OUT = 64

NC = 2
NS = 16
NW = NC * NS

CHUNK = 128
N_PAD = 10240
ROWS_PER_TILE = N_PAD // NS
CHUNKS_PER_TILE = 80
EDGES_PER_TILE = CHUNKS_PER_TILE * CHUNK
E_PAD = NW * EDGES_PER_TILE
NBUF = 4
WCHUNKS = 16
WINDOWS = CHUNKS_PER_TILE // WCHUNKS

_MESH = plsc.VectorSubcoreMesh(
    core_axis_name="c", subcore_axis_name="s", num_cores=NC, num_subcores=NS)
_PARAMS = pltpu.CompilerParams(use_tc_tiling_on_sc=False)


def _deg_body(src_ref, dst_ref, ones_ref, zeros_ref, do_out, di_out,
              sidx, didx, ones_v, sem, do_acc, di_acc):
    c = lax.axis_index("c")
    s = lax.axis_index("s")
    wid = c * NS + s
    row0 = s * ROWS_PER_TILE
    pltpu.sync_copy(zeros_ref.at[pl.ds(row0, ROWS_PER_TILE)],
                    do_acc.at[pl.ds(row0, ROWS_PER_TILE)])
    pltpu.sync_copy(zeros_ref.at[pl.ds(row0, ROWS_PER_TILE)],
                    di_acc.at[pl.ds(row0, ROWS_PER_TILE)])
    pltpu.sync_copy(ones_ref, ones_v)
    pltpu.sync_copy(src_ref.at[pl.ds(wid * CHUNKS_PER_TILE, CHUNKS_PER_TILE)], sidx)
    pltpu.sync_copy(dst_ref.at[pl.ds(wid * CHUNKS_PER_TILE, CHUNKS_PER_TILE)], didx)
    plsc.subcore_barrier()

    def issue(i, carry):
        pltpu.async_copy(ones_v, do_acc.at[sidx.at[i]], sem.at[0], add=True)
        pltpu.async_copy(ones_v, di_acc.at[didx.at[i]], sem.at[1], add=True)
        return carry

    lax.fori_loop(0, CHUNKS_PER_TILE, issue, 0)

    def drain(i, carry):
        pltpu.make_async_copy(ones_v, do_acc.at[sidx.at[i]], sem.at[0]).wait()
        pltpu.make_async_copy(ones_v, di_acc.at[didx.at[i]], sem.at[1]).wait()
        return carry

    lax.fori_loop(0, CHUNKS_PER_TILE, drain, 0)
    plsc.subcore_barrier()
    pltpu.sync_copy(do_acc.at[pl.ds(row0, ROWS_PER_TILE)],
                    do_out.at[c, pl.ds(row0, ROWS_PER_TILE)])
    pltpu.sync_copy(di_acc.at[pl.ds(row0, ROWS_PER_TILE)],
                    di_out.at[c, pl.ds(row0, ROWS_PER_TILE)])


def _make_deg_kernel():
    return pl.kernel(
        _deg_body,
        out_type=(jax.ShapeDtypeStruct((NC, N_PAD, 16), jnp.float32),
                  jax.ShapeDtypeStruct((NC, N_PAD, 16), jnp.float32)),
        mesh=_MESH,
        compiler_params=_PARAMS,
        scratch_types=[
            pltpu.VMEM((CHUNKS_PER_TILE, CHUNK), jnp.int32),
            pltpu.VMEM((CHUNKS_PER_TILE, CHUNK), jnp.int32),
            pltpu.VMEM((CHUNK, 16), jnp.float32),
            pltpu.SemaphoreType.DMA((2,)),
            pltpu.VMEM_SHARED((N_PAD, 16), jnp.float32),
            pltpu.VMEM_SHARED((N_PAD, 16), jnp.float32),
        ],
    )


def _run_edge_windows(src_ref, dst_ref, table_sp, acc, sidx, didx, rows, gsem,
                      chunk_base, n_windows):
    def gather(i, b):
        pltpu.async_copy(table_sp.at[sidx.at[i]], rows.at[b], gsem.at[b])

    def window(w, carry):
        base = chunk_base + w * WCHUNKS
        pltpu.sync_copy(src_ref.at[pl.ds(base, WCHUNKS)], sidx)
        pltpu.sync_copy(dst_ref.at[pl.ds(base, WCHUNKS)], didx)
        for b in range(NBUF):
            gather(b, b)

        def group(g, carry2):
            for b in range(NBUF):
                i = g * NBUF + b
                pltpu.make_async_copy(table_sp.at[sidx.at[i]], rows.at[b],
                                      gsem.at[b]).wait()
                pltpu.sync_copy(rows.at[b], acc.at[didx.at[i]], add=True)
                j = i + NBUF

                @pl.when(j < WCHUNKS)
                def _():
                    gather(j, b)
            return carry2

        lax.fori_loop(0, WCHUNKS // NBUF, group, 0)
        return carry

    lax.fori_loop(0, n_windows, window, 0)


def _prop_split_body(table_ref, src_ref, dst_ref, zeros_ref, out_ref,
                     sidx, didx, rows, gsem, table_sp, acc):
    c = lax.axis_index("c")
    s = lax.axis_index("s")
    row0 = s * ROWS_PER_TILE
    pltpu.sync_copy(table_ref.at[c, pl.ds(row0, ROWS_PER_TILE)],
                    table_sp.at[pl.ds(row0, ROWS_PER_TILE)])
    pltpu.sync_copy(zeros_ref.at[pl.ds(row0, ROWS_PER_TILE)],
                    acc.at[pl.ds(row0, ROWS_PER_TILE)])
    plsc.subcore_barrier()
    _run_edge_windows(src_ref, dst_ref, table_sp, acc, sidx, didx, rows, gsem,
                      s * (2 * CHUNKS_PER_TILE), 2 * WINDOWS)
    plsc.subcore_barrier()
    pltpu.sync_copy(acc.at[pl.ds(row0, ROWS_PER_TILE)],
                    out_ref.at[c, pl.ds(row0, ROWS_PER_TILE)])


def _prop_full_body(table_ref, src_ref, dst_ref, zeros_ref, out_ref,
                    sidx, didx, rows, gsem, table_sp, acc):
    c = lax.axis_index("c")
    s = lax.axis_index("s")
    wid = c * NS + s
    row0 = s * ROWS_PER_TILE
    pltpu.sync_copy(table_ref.at[pl.ds(row0, ROWS_PER_TILE)],
                    table_sp.at[pl.ds(row0, ROWS_PER_TILE)])
    pltpu.sync_copy(zeros_ref.at[pl.ds(row0, ROWS_PER_TILE)],
                    acc.at[pl.ds(row0, ROWS_PER_TILE)])
    plsc.subcore_barrier()
    _run_edge_windows(src_ref, dst_ref, table_sp, acc, sidx, didx, rows, gsem,
                      wid * CHUNKS_PER_TILE, WINDOWS)
    plsc.subcore_barrier()
    pltpu.sync_copy(acc.at[pl.ds(row0, ROWS_PER_TILE)],
                    out_ref.at[c, pl.ds(row0, ROWS_PER_TILE)])


def _make_prop_kernel(body, d, dtype):
    return pl.kernel(
        body,
        out_type=jax.ShapeDtypeStruct((NC, N_PAD, d), dtype),
        mesh=_MESH,
        compiler_params=_PARAMS,
        scratch_types=[
            pltpu.VMEM((WCHUNKS, CHUNK), jnp.int32),
            pltpu.VMEM((WCHUNKS, CHUNK), jnp.int32),
            pltpu.VMEM((NBUF, CHUNK, d), dtype),
            pltpu.SemaphoreType.DMA((NBUF,)),
            pltpu.VMEM_SHARED((N_PAD, d), dtype),
            pltpu.VMEM_SHARED((N_PAD, d), dtype),
        ],
    )


def _norm_from(parts_ref):
    deg = parts_ref[0, :, 0:1] + parts_ref[1, :, 0:1]
    return jnp.where(deg > 0.0, lax.rsqrt(jnp.maximum(deg, 1.0)), 0.0)


def _tc1_body(do_ref, feat_ref, w1_ref, y1_ref):
    nsrc = _norm_from(do_ref)
    y = jnp.dot(feat_ref[...] * nsrc, w1_ref[...],
                preferred_element_type=jnp.float32).astype(jnp.bfloat16)
    y1_ref[0] = y[:, :D_H // 2]
    y1_ref[1] = y[:, D_H // 2:]


def _tc2_body(p_ref, do_ref, di_ref, b1_ref, w2_ref, y2_ref):
    nsrc = _norm_from(do_ref)
    ndst = _norm_from(di_ref)
    agg = jnp.concatenate([p_ref[0], p_ref[1]],
                          axis=-1).astype(jnp.float32) * ndst + b1_ref[...]
    h = jnp.maximum(agg, 0.0)
    y2_ref[...] = jnp.dot(h * nsrc, w2_ref[...],
                          preferred_element_type=jnp.float32).astype(jnp.bfloat16)


def _tc3_body(p_ref, di_ref, b2_ref, o_ref):
    ndst = _norm_from(di_ref)
    p = p_ref[0].astype(jnp.float32) + p_ref[1].astype(jnp.float32)
    o_ref[...] = p * ndst + b2_ref[...]


_R = 1024
_GRID = N_PAD // _R


def _deg_spec():
    return pl.BlockSpec((NC, _R, 16), lambda i: (0, i, 0))


def kernel(features, edge_index, W1, b1, W2, b2):
    src = edge_index[0]
    dst = edge_index[1]
    pad_idx = jnp.full((E_PAD - E,), N, dtype=jnp.int32)
    src_p = jnp.concatenate([src, pad_idx]).reshape(NW * CHUNKS_PER_TILE, CHUNK)
    dst_p = jnp.concatenate([dst, pad_idx]).reshape(NW * CHUNKS_PER_TILE, CHUNK)
    feat_p = jnp.zeros((N_PAD, D_IN), jnp.float32).at[:N].set(features)
    ones16 = jnp.ones((CHUNK, 16), jnp.float32)
    zeros16 = jnp.zeros((N_PAD, 16), jnp.float32)

    do_part, di_part = _make_deg_kernel()(src_p, dst_p, ones16, zeros16)

    y1 = pl.pallas_call(
        _tc1_body,
        grid=(_GRID,),
        in_specs=[
            _deg_spec(),
            pl.BlockSpec((_R, D_IN), lambda i: (i, 0)),
            pl.BlockSpec((D_IN, D_H), lambda i: (0, 0)),
        ],
        out_specs=pl.BlockSpec((NC, _R, D_H // 2), lambda i: (0, i, 0)),
        out_shape=jax.ShapeDtypeStruct((NC, N_PAD, D_H // 2), jnp.bfloat16),
    )(do_part, feat_p, W1)

    zeros_bf = jnp.zeros((N_PAD, D_H // 2), jnp.bfloat16)
    p1 = _make_prop_kernel(_prop_split_body, D_H // 2, jnp.bfloat16)(
        y1, src_p, dst_p, zeros_bf)

    y2 = pl.pallas_call(
        _tc2_body,
        grid=(_GRID,),
        in_specs=[
            pl.BlockSpec((NC, _R, D_H // 2), lambda i: (0, i, 0)),
            _deg_spec(),
            _deg_spec(),
            pl.BlockSpec((1, D_H), lambda i: (0, 0)),
            pl.BlockSpec((D_H, D_OUT), lambda i: (0, 0)),
        ],
        out_specs=pl.BlockSpec((_R, D_OUT), lambda i: (i, 0)),
        out_shape=jax.ShapeDtypeStruct((N_PAD, D_OUT), jnp.bfloat16),
    )(p1, do_part, di_part, b1.reshape(1, D_H), W2)

    zeros_obf = jnp.zeros((N_PAD, D_OUT), jnp.bfloat16)
    p2 = _make_prop_kernel(_prop_full_body, D_OUT, jnp.bfloat16)(
        y2, src_p, dst_p, zeros_obf)

    out = pl.pallas_call(
        _tc3_body,
        grid=(_GRID,),
        in_specs=[
            pl.BlockSpec((NC, _R, D_OUT), lambda i: (0, i, 0)),
            _deg_spec(),
            pl.BlockSpec((1, D_OUT), lambda i: (0, 0)),
        ],
        out_specs=pl.BlockSpec((_R, D_OUT), lambda i: (i, 0)),
        out_shape=jax.ShapeDtypeStruct((N_PAD, D_OUT), jnp.float32),
    )(p2, di_part, b2.reshape(1, D_OUT))

    return out[:N]


# speedup vs baseline: 2.8271x; 1.0014x over previous
import jax
import jax.numpy as jnp
from jax import lax
from jax.experimental import pallas as pl
from jax.experimental.pallas import tpu as pltpu
from jax.experimental.pallas import tpu_sc as plsc

N = 10000
E = 320000
D_IN = 128
D_H = 128
D_OUT = 64

NC = 2
NS = 16
NW = NC * NS

CHUNK = 128
N_PAD = 10240
ROWS_PER_TILE = N_PAD // NS
CHUNKS_PER_TILE = 80
EDGES_PER_TILE = CHUNKS_PER_TILE * CHUNK
E_PAD = NW * EDGES_PER_TILE
NBUF = 4
WCHUNKS = 16
WINDOWS = CHUNKS_PER_TILE // WCHUNKS

_MESH = plsc.VectorSubcoreMesh(
    core_axis_name="c", subcore_axis_name="s", num_cores=NC, num_subcores=NS)
_PARAMS = pltpu.CompilerParams(use_tc_tiling_on_sc=False)


_HROWS = N_PAD // 16


def _deg_body(src_ref, dst_ref, zeros_ref, iota_ref, do_out, di_out,
              sidx, didx, iota_v, do_hist, di_hist, do_acc, di_acc):
    c = lax.axis_index("c")
    s = lax.axis_index("s")
    wid = c * NS + s
    hrow0 = s * (_HROWS // NS)
    pltpu.sync_copy(zeros_ref.at[pl.ds(hrow0, _HROWS // NS)],
                    do_acc.at[pl.ds(hrow0, _HROWS // NS)])
    pltpu.sync_copy(zeros_ref.at[pl.ds(hrow0, _HROWS // NS)],
                    di_acc.at[pl.ds(hrow0, _HROWS // NS)])
    pltpu.sync_copy(src_ref.at[pl.ds(wid * CHUNKS_PER_TILE, CHUNKS_PER_TILE)], sidx)
    pltpu.sync_copy(dst_ref.at[pl.ds(wid * CHUNKS_PER_TILE, CHUNKS_PER_TILE)], didx)
    pltpu.sync_copy(iota_ref, iota_v)
    z = jnp.zeros((16,), jnp.float32)

    def zero(r, carry):
        do_hist[r, :] = z
        di_hist[r, :] = z
        return carry

    lax.fori_loop(0, _HROWS, zero, 0)
    ones = jnp.ones((16,), jnp.float32)

    def step(i, carry):
        for k in range(CHUNK // 16):
            v = sidx[i, pl.ds(k * 16, 16)]
            plsc.addupdate_scatter(do_hist, [v >> 4, v & 15], ones)
            w = didx[i, pl.ds(k * 16, 16)]
            plsc.addupdate_scatter(di_hist, [w >> 4, w & 15], ones)
        return carry

    lax.fori_loop(0, CHUNKS_PER_TILE, step, 0)
    plsc.subcore_barrier()
    for j in range(_HROWS // CHUNK):
        pltpu.sync_copy(do_hist.at[pl.ds(j * CHUNK, CHUNK)],
                        do_acc.at[iota_v.at[j]], add=True)
        pltpu.sync_copy(di_hist.at[pl.ds(j * CHUNK, CHUNK)],
                        di_acc.at[iota_v.at[j]], add=True)
    plsc.subcore_barrier()
    pltpu.sync_copy(do_acc.at[pl.ds(hrow0, _HROWS // NS)],
                    do_out.at[c, pl.ds(hrow0, _HROWS // NS)])
    pltpu.sync_copy(di_acc.at[pl.ds(hrow0, _HROWS // NS)],
                    di_out.at[c, pl.ds(hrow0, _HROWS // NS)])


def _make_deg_kernel():
    return pl.kernel(
        _deg_body,
        out_type=(jax.ShapeDtypeStruct((NC, _HROWS, 16), jnp.float32),
                  jax.ShapeDtypeStruct((NC, _HROWS, 16), jnp.float32)),
        mesh=_MESH,
        compiler_params=pltpu.CompilerParams(use_tc_tiling_on_sc=False,
                                             needs_layout_passes=False),
        scratch_types=[
            pltpu.VMEM((CHUNKS_PER_TILE, CHUNK), jnp.int32),
            pltpu.VMEM((CHUNKS_PER_TILE, CHUNK), jnp.int32),
            pltpu.VMEM((_HROWS // CHUNK, CHUNK), jnp.int32),
            pltpu.VMEM((_HROWS, 16), jnp.float32),
            pltpu.VMEM((_HROWS, 16), jnp.float32),
            pltpu.VMEM_SHARED((_HROWS, 16), jnp.float32),
            pltpu.VMEM_SHARED((_HROWS, 16), jnp.float32),
        ],
    )


def _run_edge_windows(src_ref, dst_ref, table_sp, acc, sidx, didx, rows, gsem,
                      chunk_base, n_windows):
    def gather(i, b):
        pltpu.async_copy(table_sp.at[sidx.at[i]], rows.at[b], gsem.at[b])

    def window(w, carry):
        base = chunk_base + w * WCHUNKS
        pltpu.sync_copy(src_ref.at[pl.ds(base, WCHUNKS)], sidx)
        pltpu.sync_copy(dst_ref.at[pl.ds(base, WCHUNKS)], didx)
        for b in range(NBUF):
            gather(b, b)

        def group(g, carry2):
            for b in range(NBUF):
                i = g * NBUF + b
                pltpu.make_async_copy(table_sp.at[sidx.at[i]], rows.at[b],
                                      gsem.at[b]).wait()
                pltpu.sync_copy(rows.at[b], acc.at[didx.at[i]], add=True)
                j = i + NBUF

                @pl.when(j < WCHUNKS)
                def _():
                    gather(j, b)
            return carry2

        lax.fori_loop(0, WCHUNKS // NBUF, group, 0)
        return carry

    lax.fori_loop(0, n_windows, window, 0)


def _prop_split_body(table_ref, src_ref, dst_ref, zeros_ref, out_ref,
                     sidx, didx, rows, gsem, table_sp, acc):
    c = lax.axis_index("c")
    s = lax.axis_index("s")
    row0 = s * ROWS_PER_TILE
    pltpu.sync_copy(table_ref.at[c, pl.ds(row0, ROWS_PER_TILE)],
                    table_sp.at[pl.ds(row0, ROWS_PER_TILE)])
    pltpu.sync_copy(zeros_ref.at[pl.ds(row0, ROWS_PER_TILE)],
                    acc.at[pl.ds(row0, ROWS_PER_TILE)])
    plsc.subcore_barrier()
    _run_edge_windows(src_ref, dst_ref, table_sp, acc, sidx, didx, rows, gsem,
                      s * (2 * CHUNKS_PER_TILE), 2 * WINDOWS)
    plsc.subcore_barrier()
    pltpu.sync_copy(acc.at[pl.ds(row0, ROWS_PER_TILE)],
                    out_ref.at[c, pl.ds(row0, ROWS_PER_TILE)])


def _prop_full_body(table_ref, src_ref, dst_ref, zeros_ref, out_ref,
                    sidx, didx, rows, gsem, table_sp, acc):
    c = lax.axis_index("c")
    s = lax.axis_index("s")
    wid = c * NS + s
    row0 = s * ROWS_PER_TILE
    pltpu.sync_copy(table_ref.at[pl.ds(row0, ROWS_PER_TILE)],
                    table_sp.at[pl.ds(row0, ROWS_PER_TILE)])
    pltpu.sync_copy(zeros_ref.at[pl.ds(row0, ROWS_PER_TILE)],
                    acc.at[pl.ds(row0, ROWS_PER_TILE)])
    plsc.subcore_barrier()
    _run_edge_windows(src_ref, dst_ref, table_sp, acc, sidx, didx, rows, gsem,
                      wid * CHUNKS_PER_TILE, WINDOWS)
    plsc.subcore_barrier()
    pltpu.sync_copy(acc.at[pl.ds(row0, ROWS_PER_TILE)],
                    out_ref.at[c, pl.ds(row0, ROWS_PER_TILE)])


def _make_prop_kernel(body, d, dtype):
    return pl.kernel(
        body,
        out_type=jax.ShapeDtypeStruct((NC, N_PAD, d), dtype),
        mesh=_MESH,
        compiler_params=_PARAMS,
        scratch_types=[
            pltpu.VMEM((WCHUNKS, CHUNK), jnp.int32),
            pltpu.VMEM((WCHUNKS, CHUNK), jnp.int32),
            pltpu.VMEM((NBUF, CHUNK, d), dtype),
            pltpu.SemaphoreType.DMA((NBUF,)),
            pltpu.VMEM_SHARED((N_PAD, d), dtype),
            pltpu.VMEM_SHARED((N_PAD, d), dtype),
        ],
    )


def _norm_from(parts_ref):
    deg = parts_ref[0] + parts_ref[1]
    return jnp.where(deg > 0.0, lax.rsqrt(jnp.maximum(deg, 1.0)), 0.0)


def _tc1_body(do_ref, feat_ref, w1_ref, y1_ref):
    nsrc = _norm_from(do_ref)
    y = jnp.dot(feat_ref[...] * nsrc, w1_ref[...],
                preferred_element_type=jnp.float32).astype(jnp.bfloat16)
    y1_ref[0] = y[:, :D_H // 2]
    y1_ref[1] = y[:, D_H // 2:]


def _tc2_body(p_ref, do_ref, di_ref, b1_ref, w2_ref, y2_ref):
    nsrc = _norm_from(do_ref)
    ndst = _norm_from(di_ref)
    agg = jnp.concatenate([p_ref[0], p_ref[1]],
                          axis=-1).astype(jnp.float32) * ndst + b1_ref[...]
    h = jnp.maximum(agg, 0.0)
    y2_ref[...] = jnp.dot(h * nsrc, w2_ref[...],
                          preferred_element_type=jnp.float32).astype(jnp.bfloat16)


def _tc3_body(p_ref, di_ref, b2_ref, o_ref):
    ndst = _norm_from(di_ref)
    p = p_ref[0].astype(jnp.float32) + p_ref[1].astype(jnp.float32)
    o_ref[...] = p * ndst + b2_ref[...]


_R = 1024
_GRID = N_PAD // _R


def _deg_spec():
    return pl.BlockSpec((NC, _R, 1), lambda i: (0, i, 0))


def kernel(features, edge_index, W1, b1, W2, b2):
    src = edge_index[0]
    dst = edge_index[1]
    pad_idx = jnp.full((E_PAD - E,), N, dtype=jnp.int32)
    src_p = jnp.concatenate([src, pad_idx]).reshape(NW * CHUNKS_PER_TILE, CHUNK)
    dst_p = jnp.concatenate([dst, pad_idx]).reshape(NW * CHUNKS_PER_TILE, CHUNK)
    feat_p = jnp.zeros((N_PAD, D_IN), jnp.float32).at[:N].set(features)
    zeros_deg = jnp.zeros((_HROWS, 16), jnp.float32)
    iota_deg = jnp.arange(_HROWS, dtype=jnp.int32).reshape(_HROWS // CHUNK, CHUNK)

    do_lin, di_lin = _make_deg_kernel()(src_p, dst_p, zeros_deg, iota_deg)
    do_part = do_lin.reshape(NC, N_PAD, 1)
    di_part = di_lin.reshape(NC, N_PAD, 1)

    y1 = pl.pallas_call(
        _tc1_body,
        grid=(_GRID,),
        in_specs=[
            _deg_spec(),
            pl.BlockSpec((_R, D_IN), lambda i: (i, 0)),
            pl.BlockSpec((D_IN, D_H), lambda i: (0, 0)),
        ],
        out_specs=pl.BlockSpec((NC, _R, D_H // 2), lambda i: (0, i, 0)),
        out_shape=jax.ShapeDtypeStruct((NC, N_PAD, D_H // 2), jnp.bfloat16),
    )(do_part, feat_p, W1)

    zeros_bf = jnp.zeros((N_PAD, D_H // 2), jnp.bfloat16)
    p1 = _make_prop_kernel(_prop_split_body, D_H // 2, jnp.bfloat16)(
        y1, src_p, dst_p, zeros_bf)

    y2 = pl.pallas_call(
        _tc2_body,
        grid=(_GRID,),
        in_specs=[
            pl.BlockSpec((NC, _R, D_H // 2), lambda i: (0, i, 0)),
            _deg_spec(),
            _deg_spec(),
            pl.BlockSpec((1, D_H), lambda i: (0, 0)),
            pl.BlockSpec((D_H, D_OUT), lambda i: (0, 0)),
        ],
        out_specs=pl.BlockSpec((_R, D_OUT), lambda i: (i, 0)),
        out_shape=jax.ShapeDtypeStruct((N_PAD, D_OUT), jnp.bfloat16),
    )(p1, do_part, di_part, b1.reshape(1, D_H), W2)

    zeros_obf = jnp.zeros((N_PAD, D_OUT), jnp.bfloat16)
    p2 = _make_prop_kernel(_prop_full_body, D_OUT, jnp.bfloat16)(
        y2, src_p, dst_p, zeros_obf)

    out = pl.pallas_call(
        _tc3_body,
        grid=(_GRID,),
        in_specs=[
            pl.BlockSpec((NC, _R, D_OUT), lambda i: (0, i, 0)),
            _deg_spec(),
            pl.BlockSpec((1, D_OUT), lambda i: (0, 0)),
        ],
        out_specs=pl.BlockSpec((_R, D_OUT), lambda i: (i, 0)),
        out_shape=jax.ShapeDtypeStruct((N_PAD, D_OUT), jnp.float32),
    )(p2, di_part, b2.reshape(1, D_OUT))

    return out[:N]
